# baseline (speedup 1.0000x reference)
import functools

import jax
import jax.numpy as jnp
from jax import lax
from jax.experimental import pallas as pl
from jax.experimental.pallas import tpu as pltpu
from jax.experimental.pallas import tpu_sc as plsc

NC = 2
NS = 16
CH = 128


def _xnorm_body(x_ref, norm_ref, o_ref):
    o_ref[...] = x_ref[...] * norm_ref[...]


def _mm_body(xnorm_ref, norm_ref, parts_ref, w_ref, b_ref, o_ref):
    xn = xnorm_ref[...]
    p = parts_ref[...]
    agg = (p[0] + p[1]) - xn
    t = agg * norm_ref[...]
    acc = jnp.dot(t, w_ref[...], preferred_element_type=jnp.float32)
    o_ref[...] = jnp.maximum(acc + b_ref[...], 0.0)


def _make_agg_kernel(n, n_pad, d, cpt):
    mesh = plsc.VectorSubcoreMesh(
        core_axis_name="c", subcore_axis_name="s", num_cores=NC,
        num_subcores=NS)
    rpt = (n // (8 * NS)) * 8
    rem = n - rpt * NS

    cpta = cpt

    @functools.partial(
        pl.kernel,
        out_type=jax.ShapeDtypeStruct((NC, n, d), jnp.float32),
        mesh=mesh,
        scratch_types=[
            pltpu.VMEM_SHARED((n_pad, d), jnp.float32),
            pltpu.VMEM((CH,), jnp.int32),
            pltpu.VMEM((CH,), jnp.int32),
            pltpu.VMEM((CH,), jnp.int32),
            pltpu.VMEM((CH,), jnp.int32),
            pltpu.VMEM((CH, d), jnp.float32),
            pltpu.VMEM((CH, d), jnp.float32),
            pltpu.SemaphoreType.DMA,
            pltpu.SemaphoreType.DMA,
        ],
    )
    def agg(xnorm_hbm, src_hbm, dst_hbm, out_hbm, acc, src0, src1,
            dst0, dst1, rows0, rows1, gsem, ssem):
        c = lax.axis_index("c")
        s = lax.axis_index("s")
        wid = c * NS + s

        pltpu.sync_copy(xnorm_hbm.at[pl.ds(s * rpt, rpt)],
                        acc.at[pl.ds(s * rpt, rpt)])
        if rem:
            @pl.when(s == 0)
            def _():
                pltpu.sync_copy(xnorm_hbm.at[pl.ds(NS * rpt, rem)],
                                acc.at[pl.ds(NS * rpt, rem)])
        plsc.subcore_barrier()

        srcs = (src0, src1)
        dsts = (dst0, dst1)
        rows = (rows0, rows1)

        def load_and_gather(i, b):
            base = (wid * cpta + i) * CH
            pltpu.sync_copy(src_hbm.at[pl.ds(base, CH)], srcs[b])
            pltpu.sync_copy(dst_hbm.at[pl.ds(base, CH)], dsts[b])
            pltpu.async_copy(xnorm_hbm.at[srcs[b]], rows[b], gsem).wait()

        def fire_scatter(b):
            pltpu.async_copy(rows[b], acc.at[dsts[b]], ssem, add=True)

        def wait_scatter(b):
            pltpu.make_async_copy(rows[b], acc.at[dsts[b]], ssem).wait()

        for b in range(2):
            load_and_gather(b, b)
            fire_scatter(b)

        def body(j, carry):
            for b in range(2):
                wait_scatter(b)
                load_and_gather(2 * j + b, b)
                fire_scatter(b)
            return carry

        lax.fori_loop(1, cpt // 2, body, 0)
        wait_scatter(0)
        wait_scatter(1)

        plsc.subcore_barrier()
        pltpu.sync_copy(acc.at[pl.ds(s * rpt, rpt)],
                        out_hbm.at[c, pl.ds(s * rpt, rpt)])
        if rem:
            @pl.when(s == 0)
            def _():
                pltpu.sync_copy(acc.at[pl.ds(NS * rpt, rem)],
                                out_hbm.at[c, pl.ds(NS * rpt, rem)])

    return agg


def kernel(X, ref_a, ref_b, norm, W, b):
    n, d = X.shape
    e = ref_a.shape[0]
    units = W.shape[1]

    n_pad = n + 512
    nw = NC * NS
    e2 = 2 * e
    cpt = -(-e2 // (nw * CH))
    cpt += cpt % 2
    pad = cpt * nw * CH - e2
    ppt = pad // nw

    n_scrap = 512
    ra = ref_a.astype(jnp.int32)
    rb = ref_b.astype(jnp.int32)
    pad_src = jnp.zeros((nw, ppt), jnp.int32)
    pad_dst = n + (jnp.arange(nw * ppt, dtype=jnp.int32)
                   % n_scrap).reshape(nw, ppt)
    src = jnp.concatenate(
        [jnp.concatenate([ra, rb]).reshape(nw, e2 // nw), pad_src],
        axis=1).reshape(-1)
    dst = jnp.concatenate(
        [jnp.concatenate([rb, ra]).reshape(nw, e2 // nw), pad_dst],
        axis=1).reshape(-1)

    bm = 1000
    grid = n // bm

    xnorm = pl.pallas_call(
        _xnorm_body,
        grid=(grid,),
        in_specs=[
            pl.BlockSpec((bm, d), lambda i: (i, 0)),
            pl.BlockSpec((bm, 1), lambda i: (i, 0)),
        ],
        out_specs=pl.BlockSpec((bm, d), lambda i: (i, 0)),
        out_shape=jax.ShapeDtypeStruct((n, d), jnp.float32),
    )(X, norm)

    parts = _make_agg_kernel(n, n_pad, d, cpt)(xnorm, src, dst)

    b2 = b.reshape(1, units)
    out = pl.pallas_call(
        _mm_body,
        grid=(grid,),
        in_specs=[
            pl.BlockSpec((bm, d), lambda i: (i, 0)),
            pl.BlockSpec((bm, 1), lambda i: (i, 0)),
            pl.BlockSpec((NC, bm, d), lambda i: (0, i, 0)),
            pl.BlockSpec((d, units), lambda i: (0, 0)),
            pl.BlockSpec((1, units), lambda i: (0, 0)),
        ],
        out_specs=pl.BlockSpec((bm, units), lambda i: (i, 0)),
        out_shape=jax.ShapeDtypeStruct((n, units), jnp.float32),
    )(xnorm, norm, parts, W, b2)

    return out


# speedup vs baseline: 1.2088x; 1.2088x over previous
import functools

import jax
import jax.numpy as jnp
from jax import lax
from jax.experimental import pallas as pl
from jax.experimental.pallas import tpu as pltpu
from jax.experimental.pallas import tpu_sc as plsc

NC = 2
NS = 16
CH = 128


def _xnorm_body(x_ref, norm_ref, o_ref):
    o_ref[...] = x_ref[...] * norm_ref[...]


def _mm_body(xnorm_ref, norm_ref, parts_ref, w_ref, b_ref, o_ref):
    xn = xnorm_ref[...]
    p = parts_ref[...]
    agg = (p[0] + p[1]) - xn
    t = agg * norm_ref[...]
    acc = jnp.dot(t, w_ref[...], preferred_element_type=jnp.float32)
    o_ref[...] = jnp.maximum(acc + b_ref[...], 0.0)


def _make_agg_kernel(n, n_pad, d, cpt):
    mesh = plsc.VectorSubcoreMesh(
        core_axis_name="c", subcore_axis_name="s", num_cores=NC,
        num_subcores=NS)
    rpt = (n // (8 * NS)) * 8
    rem = n - rpt * NS

    cpta = cpt

    @functools.partial(
        pl.kernel,
        out_type=jax.ShapeDtypeStruct((NC, n, d), jnp.float32),
        mesh=mesh,
        scratch_types=[
            pltpu.VMEM_SHARED((n_pad, d), jnp.float32),
            pltpu.VMEM((2 * CH,), jnp.int32),
            pltpu.VMEM((CH, d), jnp.float32),
            pltpu.SemaphoreType.DMA,
        ],
    )
    def agg(xnorm_hbm, sd_hbm, out_hbm, acc, sd_v, rows0, gsem):
        c = lax.axis_index("c")
        s = lax.axis_index("s")
        wid = c * NS + s

        pltpu.sync_copy(xnorm_hbm.at[pl.ds(s * rpt, rpt)],
                        acc.at[pl.ds(s * rpt, rpt)])
        if rem:
            @pl.when(s == 0)
            def _():
                pltpu.sync_copy(xnorm_hbm.at[pl.ds(NS * rpt, rem)],
                                acc.at[pl.ds(NS * rpt, rem)])
        plsc.subcore_barrier()

        def body(i, carry):
            base = (wid * cpta + i) * 2 * CH
            pltpu.sync_copy(sd_hbm.at[pl.ds(base, 2 * CH)], sd_v)
            pltpu.async_copy(xnorm_hbm.at[sd_v.at[pl.ds(0, CH)]],
                             rows0, gsem).wait()
            pltpu.sync_copy(rows0, acc.at[sd_v.at[pl.ds(CH, CH)]],
                            add=True)
            return carry

        lax.fori_loop(0, cpt, body, 0)

        plsc.subcore_barrier()
        pltpu.sync_copy(acc.at[pl.ds(s * rpt, rpt)],
                        out_hbm.at[c, pl.ds(s * rpt, rpt)])
        if rem:
            @pl.when(s == 0)
            def _():
                pltpu.sync_copy(acc.at[pl.ds(NS * rpt, rem)],
                                out_hbm.at[c, pl.ds(NS * rpt, rem)])

    return agg


def kernel(X, ref_a, ref_b, norm, W, b):
    n, d = X.shape
    e = ref_a.shape[0]
    units = W.shape[1]

    n_pad = n + 512
    nw = NC * NS
    e2 = 2 * e
    cpt = -(-e2 // (nw * CH))
    pad = cpt * nw * CH - e2
    ppt = pad // nw

    n_scrap = 512
    ra = ref_a.astype(jnp.int32)
    rb = ref_b.astype(jnp.int32)
    pad_src = jnp.zeros((nw, ppt), jnp.int32)
    pad_dst = n + (jnp.arange(nw * ppt, dtype=jnp.int32)
                   % n_scrap).reshape(nw, ppt)
    src = jnp.concatenate(
        [jnp.concatenate([ra, rb]).reshape(nw, e2 // nw), pad_src],
        axis=1).reshape(nw * cpt, CH)
    dst = jnp.concatenate(
        [jnp.concatenate([rb, ra]).reshape(nw, e2 // nw), pad_dst],
        axis=1).reshape(nw * cpt, CH)
    sd = jnp.stack([src, dst], axis=1).reshape(-1)

    bm = 1000
    grid = n // bm

    xnorm = pl.pallas_call(
        _xnorm_body,
        grid=(grid,),
        in_specs=[
            pl.BlockSpec((bm, d), lambda i: (i, 0)),
            pl.BlockSpec((bm, 1), lambda i: (i, 0)),
        ],
        out_specs=pl.BlockSpec((bm, d), lambda i: (i, 0)),
        out_shape=jax.ShapeDtypeStruct((n, d), jnp.float32),
    )(X, norm)

    parts = _make_agg_kernel(n, n_pad, d, cpt)(xnorm, sd)

    b2 = b.reshape(1, units)
    out = pl.pallas_call(
        _mm_body,
        grid=(grid,),
        in_specs=[
            pl.BlockSpec((bm, d), lambda i: (i, 0)),
            pl.BlockSpec((bm, 1), lambda i: (i, 0)),
            pl.BlockSpec((NC, bm, d), lambda i: (0, i, 0)),
            pl.BlockSpec((d, units), lambda i: (0, 0)),
            pl.BlockSpec((1, units), lambda i: (0, 0)),
        ],
        out_specs=pl.BlockSpec((bm, units), lambda i: (i, 0)),
        out_shape=jax.ShapeDtypeStruct((n, units), jnp.float32),
    )(xnorm, norm, parts, W, b2)

    return out
